# baseline (speedup 1.0000x reference)
import functools

import jax
import jax.numpy as jnp
from jax import lax
from jax.experimental import pallas as pl
from jax.experimental.pallas import tpu as pltpu
from jax.experimental.pallas import tpu_sc as plsc

_TOP_K = 2
_SCALE = 2.5
_R = 512
_D = 8
_L = 16


def _score_body(hs_hbm, w_ref, sc_hbm, buf, sc_acc, sem, osem):
    i = pl.program_id(0)
    nsteps = pl.num_programs(0)

    @pl.when(i == 0)
    def _prime():
        for j in range(_D):
            pltpu.make_async_copy(
                hs_hbm.at[pl.ds(j * _R, _R)], buf.at[j], sem.at[j]
            ).start()

    slot = jax.lax.rem(i, _D)
    pltpu.make_async_copy(
        hs_hbm.at[pl.ds(i * _R, _R)], buf.at[slot], sem.at[slot]
    ).wait()
    hs = buf[slot]

    logits_t = jax.lax.dot_general(
        w_ref[...], hs, (((1,), (1,)), ((), ())), preferred_element_type=jnp.float32
    )
    sc_acc[:, pl.ds(i * _R, _R)] = jax.nn.sigmoid(logits_t)

    nxt = i + _D

    @pl.when(nxt < nsteps)
    def _refill():
        pltpu.make_async_copy(
            hs_hbm.at[pl.ds(nxt * _R, _R)], buf.at[slot], sem.at[slot]
        ).start()

    @pl.when(i == nsteps - 1)
    def _flush():
        cp = pltpu.make_async_copy(sc_acc, sc_hbm, osem)
        cp.start()
        cp.wait()


@functools.lru_cache(maxsize=None)
def _make_sc_router(n, e):
    info = plsc.get_sparse_core_info()
    nc, ns = info.num_cores, info.num_subcores
    nw = nc * ns
    bpw = n // nw
    mesh = plsc.VectorSubcoreMesh(core_axis_name="c", subcore_axis_name="s")

    @functools.partial(
        pl.kernel,
        mesh=mesh,
        out_type=[
            jax.ShapeDtypeStruct((_TOP_K, n), jnp.int32),
            jax.ShapeDtypeStruct((_TOP_K, n), jnp.float32),
        ],
        scratch_types=[
            pltpu.VMEM((e, bpw), jnp.float32),
            pltpu.VMEM((_TOP_K, bpw), jnp.int32),
            pltpu.VMEM((_TOP_K, bpw), jnp.float32),
            pltpu.SemaphoreType.DMA,
        ],
    )
    def router(sc_hbm, idx_hbm, wt_hbm, sc_v, idx_v, wt_v, sem):
        wid = lax.axis_index("s") * nc + lax.axis_index("c")
        base = wid * bpw
        loads = [
            pltpu.async_copy(sc_hbm.at[ee, pl.ds(base, bpw)], sc_v.at[ee], sem)
            for ee in range(e)
        ]
        for cp in loads:
            cp.wait()

        def chunk(c, carry):
            off = c * _L
            m1 = sc_v[0, pl.ds(off, _L)]
            i1 = jnp.zeros((_L,), jnp.int32)
            m2 = jnp.full((_L,), -1.0, jnp.float32)
            i2 = jnp.zeros((_L,), jnp.int32)
            for ee in range(1, e):
                s = sc_v[ee, pl.ds(off, _L)]
                ev = jnp.full((_L,), ee, jnp.int32)
                gt1 = s > m1
                dm = jnp.where(gt1, m1, s)
                di = jnp.where(gt1, i1, ev)
                m1 = jnp.where(gt1, s, m1)
                i1 = jnp.where(gt1, ev, i1)
                gt2 = dm > m2
                m2 = jnp.where(gt2, dm, m2)
                i2 = jnp.where(gt2, di, i2)
            inv = _SCALE / (m1 + m2 + 1e-20)
            idx_v[0, pl.ds(off, _L)] = i1
            idx_v[1, pl.ds(off, _L)] = i2
            wt_v[0, pl.ds(off, _L)] = m1 * inv
            wt_v[1, pl.ds(off, _L)] = m2 * inv
            return carry

        lax.fori_loop(0, bpw // _L, chunk, 0)

        stores = [
            pltpu.async_copy(idx_v.at[t], idx_hbm.at[t, pl.ds(base, bpw)], sem)
            for t in range(_TOP_K)
        ] + [
            pltpu.async_copy(wt_v.at[t], wt_hbm.at[t, pl.ds(base, bpw)], sem)
            for t in range(_TOP_K)
        ]
        for cp in stores:
            cp.wait()

    return router


def kernel(hidden_states, weight):
    b, s, h = hidden_states.shape
    n = b * s
    e = weight.shape[0]
    hs = hidden_states.reshape(n, h)
    scores_t = pl.pallas_call(
        _score_body,
        grid=(n // _R,),
        in_specs=[
            pl.BlockSpec(memory_space=pl.ANY),
            pl.BlockSpec((e, h), lambda i: (0, 0)),
        ],
        out_specs=pl.BlockSpec(memory_space=pl.ANY),
        out_shape=jax.ShapeDtypeStruct((e, n), jnp.float32),
        scratch_shapes=[
            pltpu.VMEM((_D, _R, 2048), jnp.float32),
            pltpu.VMEM((e, n), jnp.float32),
            pltpu.SemaphoreType.DMA((_D,)),
            pltpu.SemaphoreType.DMA,
        ],
    )(hs, weight)
    idx_t, wt_t = _make_sc_router(n, e)(scores_t)
    return idx_t.T, wt_t.T
